# baseline (speedup 1.0000x reference)
import functools

import jax
import jax.numpy as jnp
from jax import lax
from jax.experimental import pallas as pl
from jax.experimental.pallas import tpu as pltpu
from jax.experimental.pallas import tpu_sc as plsc

B = 16384
NW = 32
BPW = B // NW
C = 64
NCH = BPW // C
L = 16

ITEM_ROWS, ITEM_D, ITEM_BLK = 1000000, 64, 2048
BRAND_ROWS, BRAND_D, BRAND_BLK = 100000, 32, 1024
CAT_ROWS, CAT_D, CAT_BLK = 1000, 32, 256


def _mk_pack(rows, d, blk):
    p = 128 // d
    tr = (rows + p * blk - 1) // (p * blk)
    h = tr * blk
    last = (rows - 1) // blk

    def body(*refs):
        x = jnp.concatenate([r[...] for r in refs[:-1]], axis=0)
        eye = jnp.eye(128, dtype=jnp.float32)
        dn = (((0,), (0,)), ((), ()))
        refs[-1][...] = lax.dot_general(x, eye, dn,
                                        preferred_element_type=jnp.float32)

    in_specs = [
        pl.BlockSpec((d, blk),
                     lambda c, k=k: (0, jnp.minimum(c + k * tr, last)))
        for k in range(p)
    ]

    def pack(tbl_t):
        return pl.pallas_call(
            body,
            grid=(tr,),
            in_specs=in_specs,
            out_specs=pl.BlockSpec((blk, 128), lambda c: (c, 0)),
            out_shape=jax.ShapeDtypeStruct((h, 128), jnp.float32),
        )(*([tbl_t] * p))

    return pack, h


_pack_item, H_ITEM = _mk_pack(ITEM_ROWS, ITEM_D, ITEM_BLK)
_pack_brand, H_BRAND = _mk_pack(BRAND_ROWS, BRAND_D, BRAND_BLK)
_pack_cat, H_CAT = _mk_pack(CAT_ROWS, CAT_D, CAT_BLK)


def _sc_body(u_idx, pi_idx, po_idx, pc_idx, pco_idx, pb_idx, pbo_idx,
             ni_idx, no_idx, nc_idx, nco_idx, nb_idx, nbo_idx,
             user_table, item_pk, cat_pk, brand_pk,
             out_hbm,
             idx_v, u_rows, pi_rows, pc_rows, pb_rows,
             ni_rows, nc_rows, nb_rows, partial,
             sem):
    nc_ax = jax.lax.axis_index("c")
    ns_ax = jax.lax.axis_index("s")
    wid = ns_ax * 2 + nc_ax
    base = wid * BPW

    def chunk_body(c, _):
        pltpu.sync_copy(u_idx.at[wid, c], idx_v.at[0])
        pltpu.sync_copy(pi_idx.at[wid, c], idx_v.at[1])
        pltpu.sync_copy(po_idx.at[wid, c], idx_v.at[2])
        pltpu.sync_copy(pc_idx.at[wid, c], idx_v.at[3])
        pltpu.sync_copy(pco_idx.at[wid, c], idx_v.at[4])
        pltpu.sync_copy(pb_idx.at[wid, c], idx_v.at[5])
        pltpu.sync_copy(pbo_idx.at[wid, c], idx_v.at[6])
        pltpu.sync_copy(ni_idx.at[wid, c], idx_v.at[7])
        pltpu.sync_copy(no_idx.at[wid, c], idx_v.at[8])
        pltpu.sync_copy(nc_idx.at[wid, c], idx_v.at[9])
        pltpu.sync_copy(nco_idx.at[wid, c], idx_v.at[10])
        pltpu.sync_copy(nb_idx.at[wid, c], idx_v.at[11])
        pltpu.sync_copy(nbo_idx.at[wid, c], idx_v.at[12])

        cps = [
            pltpu.async_copy(user_table.at[idx_v.at[0]], u_rows, sem),
            pltpu.async_copy(item_pk.at[idx_v.at[1]], pi_rows, sem),
            pltpu.async_copy(cat_pk.at[idx_v.at[3]], pc_rows, sem),
            pltpu.async_copy(brand_pk.at[idx_v.at[5]], pb_rows, sem),
            pltpu.async_copy(item_pk.at[idx_v.at[7]], ni_rows, sem),
            pltpu.async_copy(cat_pk.at[idx_v.at[9]], nc_rows, sem),
            pltpu.async_copy(brand_pk.at[idx_v.at[11]], nb_rows, sem),
        ]
        for cp in cps:
            cp.wait()

        def group_body(g, _):
            b16 = g * L
            d16 = pl.ds(b16, L)
            pov = idx_v[2, d16]
            pcov = idx_v[4, d16]
            pbov = idx_v[6, d16]
            nov = idx_v[8, d16]
            ncov = idx_v[10, d16]
            nbov = idx_v[12, d16]
            for j in range(L):
                r = b16 + j
                po_j = pov[j]
                pco_j = pcov[j]
                pbo_j = pbov[j]
                no_j = nov[j]
                nco_j = ncov[j]
                nbo_j = nbov[j]
                acc = jnp.zeros((L,), jnp.float32)
                for k in range(4):
                    du = pl.ds(k * L, L)
                    pv = pi_rows[r, pl.ds(po_j + k * L, L)]
                    nv = ni_rows[r, pl.ds(no_j + k * L, L)]
                    acc += u_rows[r, du] * (nv - pv)
                for k in range(2):
                    du = pl.ds(64 + k * L, L)
                    pv = pc_rows[r, pl.ds(pco_j + k * L, L)]
                    nv = nc_rows[r, pl.ds(nco_j + k * L, L)]
                    acc += u_rows[r, du] * (nv - pv)
                for k in range(2):
                    du = pl.ds(96 + k * L, L)
                    pv = pb_rows[r, pl.ds(pbo_j + k * L, L)]
                    nv = nb_rows[r, pl.ds(nbo_j + k * L, L)]
                    acc += u_rows[r, du] * (nv - pv)
                partial[r, :] = acc
            return 0

        lax.fori_loop(0, C // L, group_body, 0)
        pltpu.sync_copy(partial, out_hbm.at[pl.ds(base + c * C, C)])
        return 0

    lax.fori_loop(0, NCH, chunk_body, 0)


@jax.jit
def _sc_scores(*args):
    mesh = plsc.VectorSubcoreMesh(core_axis_name="c", subcore_axis_name="s")
    f = functools.partial(
        pl.kernel,
        mesh=mesh,
        compiler_params=pltpu.CompilerParams(needs_layout_passes=False),
        out_type=jax.ShapeDtypeStruct((B, L), jnp.float32),
        scratch_types=[
            pltpu.VMEM((13, C), jnp.int32),
            pltpu.VMEM((C, 128), jnp.float32),
            pltpu.VMEM((C, 128), jnp.float32),
            pltpu.VMEM((C, 128), jnp.float32),
            pltpu.VMEM((C, 128), jnp.float32),
            pltpu.VMEM((C, 128), jnp.float32),
            pltpu.VMEM((C, 128), jnp.float32),
            pltpu.VMEM((C, 128), jnp.float32),
            pltpu.VMEM((C, L), jnp.float32),
            pltpu.SemaphoreType.DMA,
        ],
    )(_sc_body)
    return f(*args)


def _tc_loss_body(p_ref, o_ref):
    s = jnp.sum(p_ref[...], axis=1, keepdims=True)
    sp = jnp.maximum(s, 0.0) + jnp.log1p(jnp.exp(-jnp.abs(s)))
    o_ref[...] = (jnp.sum(sp) * (1.0 / B)).reshape(1, 1)


def _tc_loss(partials):
    out = pl.pallas_call(
        _tc_loss_body,
        out_shape=jax.ShapeDtypeStruct((1, 1), jnp.float32),
    )(partials)
    return out[0, 0]


def _split(idx, h, d):
    q = idx // h
    return idx - q * h, q * d


def kernel(user, item, item_cat, item_brand, neg_item, neg_item_cat,
           neg_item_brand, user_table, item_table, cat_table, brand_table):
    def rs(x):
        return x.astype(jnp.int32).reshape(NW, NCH, C)

    item_pk = _pack_item(item_table.T)
    cat_pk = _pack_cat(cat_table.T)
    brand_pk = _pack_brand(brand_table.T)

    pi, po = _split(item.astype(jnp.int32), H_ITEM, ITEM_D)
    pc, pco = _split(item_cat.astype(jnp.int32), H_CAT, CAT_D)
    pb, pbo = _split(item_brand.astype(jnp.int32), H_BRAND, BRAND_D)
    ni, no = _split(neg_item.astype(jnp.int32), H_ITEM, ITEM_D)
    ncc, nco = _split(neg_item_cat.astype(jnp.int32), H_CAT, CAT_D)
    nb, nbo = _split(neg_item_brand.astype(jnp.int32), H_BRAND, BRAND_D)

    partials = _sc_scores(
        rs(user), rs(pi), rs(po), rs(pc), rs(pco), rs(pb), rs(pbo),
        rs(ni), rs(no), rs(ncc), rs(nco), rs(nb), rs(nbo),
        user_table, item_pk, cat_pk, brand_pk)
    return _tc_loss(partials)


# speedup vs baseline: 1.3539x; 1.3539x over previous
import functools

import jax
import jax.numpy as jnp
from jax import lax
from jax.experimental import pallas as pl
from jax.experimental.pallas import tpu as pltpu
from jax.experimental.pallas import tpu_sc as plsc

B = 16384
NW = 32
BPW = B // NW
C = 64
NCH = BPW // C
L = 16

ITEM_ROWS, ITEM_D, ITEM_BLK = 1000000, 64, 4096
BRAND_ROWS, BRAND_D, BRAND_BLK = 100000, 32, 1024
CAT_ROWS, CAT_D, CAT_BLK = 1000, 32, 256


def _mk_pack(rows, d, blk):
    p = 128 // d
    tr = (rows + p * blk - 1) // (p * blk)
    h = tr * blk
    last = (rows - 1) // blk

    def body(*refs):
        x = jnp.concatenate([r[...] for r in refs[:-1]], axis=0)
        eye = jnp.eye(128, dtype=jnp.float32)
        dn = (((0,), (0,)), ((), ()))
        refs[-1][...] = lax.dot_general(x, eye, dn,
                                        preferred_element_type=jnp.float32)

    in_specs = [
        pl.BlockSpec((d, blk),
                     lambda c, k=k: (0, jnp.minimum(c + k * tr, last)))
        for k in range(p)
    ]

    def pack(tbl_t):
        return pl.pallas_call(
            body,
            grid=(tr,),
            in_specs=in_specs,
            out_specs=pl.BlockSpec((blk, 128), lambda c: (c, 0)),
            out_shape=jax.ShapeDtypeStruct((h, 128), jnp.float32),
        )(*([tbl_t] * p))

    return pack, h


_pack_item, H_ITEM = _mk_pack(ITEM_ROWS, ITEM_D, ITEM_BLK)
_pack_brand, H_BRAND = _mk_pack(BRAND_ROWS, BRAND_D, BRAND_BLK)
_pack_cat, H_CAT = _mk_pack(CAT_ROWS, CAT_D, CAT_BLK)


def _sc_body(all_idx,
             user_table, item_pk, cat_pk, brand_pk,
             out_hbm,
             idx_v, u_rows, pi_rows, pc_rows, pb_rows,
             ni_rows, nc_rows, nb_rows, partial,
             sem):
    nc_ax = jax.lax.axis_index("c")
    ns_ax = jax.lax.axis_index("s")
    wid = ns_ax * 2 + nc_ax
    base = wid * BPW

    def chunk_body(c, _):
        pltpu.sync_copy(all_idx.at[wid, c], idx_v)
        cps = [
            pltpu.async_copy(user_table.at[idx_v.at[0]], u_rows, sem),
            pltpu.async_copy(item_pk.at[idx_v.at[1]], pi_rows, sem),
            pltpu.async_copy(cat_pk.at[idx_v.at[3]], pc_rows, sem),
            pltpu.async_copy(brand_pk.at[idx_v.at[5]], pb_rows, sem),
            pltpu.async_copy(item_pk.at[idx_v.at[7]], ni_rows, sem),
            pltpu.async_copy(cat_pk.at[idx_v.at[9]], nc_rows, sem),
            pltpu.async_copy(brand_pk.at[idx_v.at[11]], nb_rows, sem),
        ]
        for cp in cps:
            cp.wait()

        def group_body(g, _):
            b16 = g * L
            d16 = pl.ds(b16, L)
            pov = idx_v[2, d16]
            pcov = idx_v[4, d16]
            pbov = idx_v[6, d16]
            nov = idx_v[8, d16]
            ncov = idx_v[10, d16]
            nbov = idx_v[12, d16]
            for j in range(L):
                r = b16 + j
                po_j = pov[j]
                pco_j = pcov[j]
                pbo_j = pbov[j]
                no_j = nov[j]
                nco_j = ncov[j]
                nbo_j = nbov[j]
                acc = jnp.zeros((L,), jnp.float32)
                for k in range(4):
                    du = pl.ds(k * L, L)
                    pv = pi_rows[r, pl.ds(po_j + k * L, L)]
                    nv = ni_rows[r, pl.ds(no_j + k * L, L)]
                    acc += u_rows[r, du] * (nv - pv)
                for k in range(2):
                    du = pl.ds(64 + k * L, L)
                    pv = pc_rows[r, pl.ds(pco_j + k * L, L)]
                    nv = nc_rows[r, pl.ds(nco_j + k * L, L)]
                    acc += u_rows[r, du] * (nv - pv)
                for k in range(2):
                    du = pl.ds(96 + k * L, L)
                    pv = pb_rows[r, pl.ds(pbo_j + k * L, L)]
                    nv = nb_rows[r, pl.ds(nbo_j + k * L, L)]
                    acc += u_rows[r, du] * (nv - pv)
                partial[r, :] = acc
            return 0

        lax.fori_loop(0, C // L, group_body, 0)
        pltpu.sync_copy(partial, out_hbm.at[pl.ds(base + c * C, C)])
        return 0

    lax.fori_loop(0, NCH, chunk_body, 0)


@jax.jit
def _sc_scores(*args):
    mesh = plsc.VectorSubcoreMesh(core_axis_name="c", subcore_axis_name="s")
    f = functools.partial(
        pl.kernel,
        mesh=mesh,
        compiler_params=pltpu.CompilerParams(needs_layout_passes=False),
        out_type=jax.ShapeDtypeStruct((B, L), jnp.float32),
        scratch_types=[
            pltpu.VMEM((13, C), jnp.int32),
            pltpu.VMEM((C, 128), jnp.float32),
            pltpu.VMEM((C, 128), jnp.float32),
            pltpu.VMEM((C, 128), jnp.float32),
            pltpu.VMEM((C, 128), jnp.float32),
            pltpu.VMEM((C, 128), jnp.float32),
            pltpu.VMEM((C, 128), jnp.float32),
            pltpu.VMEM((C, 128), jnp.float32),
            pltpu.VMEM((C, L), jnp.float32),
            pltpu.SemaphoreType.DMA,
        ],
    )(_sc_body)
    return f(*args)


def _tc_loss_body(p_ref, o_ref):
    s = jnp.sum(p_ref[...], axis=1, keepdims=True)
    sp = jnp.maximum(s, 0.0) + jnp.log1p(jnp.exp(-jnp.abs(s)))
    o_ref[...] = (jnp.sum(sp) * (1.0 / B)).reshape(1, 1)


def _tc_loss(partials):
    out = pl.pallas_call(
        _tc_loss_body,
        out_shape=jax.ShapeDtypeStruct((1, 1), jnp.float32),
    )(partials)
    return out[0, 0]


def _split(idx, h, d):
    q = idx // h
    return idx - q * h, q * d


def kernel(user, item, item_cat, item_brand, neg_item, neg_item_cat,
           neg_item_brand, user_table, item_table, cat_table, brand_table):
    def rs(x):
        return x.astype(jnp.int32).reshape(NW, NCH, C)

    item_pk = _pack_item(item_table.T)
    cat_pk = _pack_cat(cat_table.T)
    brand_pk = _pack_brand(brand_table.T)

    pi, po = _split(item.astype(jnp.int32), H_ITEM, ITEM_D)
    pc, pco = _split(item_cat.astype(jnp.int32), H_CAT, CAT_D)
    pb, pbo = _split(item_brand.astype(jnp.int32), H_BRAND, BRAND_D)
    ni, no = _split(neg_item.astype(jnp.int32), H_ITEM, ITEM_D)
    ncc, nco = _split(neg_item_cat.astype(jnp.int32), H_CAT, CAT_D)
    nb, nbo = _split(neg_item_brand.astype(jnp.int32), H_BRAND, BRAND_D)

    all_idx = jnp.stack(
        [rs(user), rs(pi), rs(po), rs(pc), rs(pco), rs(pb), rs(pbo),
         rs(ni), rs(no), rs(ncc), rs(nco), rs(nb), rs(nbo)],
        axis=2)
    partials = _sc_scores(all_idx, user_table, item_pk, cat_pk, brand_pk)
    return _tc_loss(partials)


# speedup vs baseline: 1.5320x; 1.1316x over previous
import functools

import jax
import jax.numpy as jnp
from jax import lax
from jax.experimental import pallas as pl
from jax.experimental.pallas import tpu as pltpu
from jax.experimental.pallas import tpu_sc as plsc

B = 16384
NW = 32
BPW = B // NW
C = 64
NCH = BPW // C
L = 16

ITEM_ROWS, ITEM_D, ITEM_BLK = 1000000, 64, 8192
BRAND_ROWS, BRAND_D, BRAND_BLK = 100000, 32, 4096
CAT_ROWS, CAT_D, CAT_BLK = 1000, 32, 256


def _mk_pack(rows, d, blk):
    p = 128 // d
    tr = (rows + p * blk - 1) // (p * blk)
    h = tr * blk
    last = (rows - 1) // blk

    def body(*refs):
        x = jnp.concatenate([r[...] for r in refs[:-1]], axis=0)
        eye = jnp.eye(128, dtype=jnp.float32)
        dn = (((0,), (0,)), ((), ()))
        refs[-1][...] = lax.dot_general(x, eye, dn,
                                        preferred_element_type=jnp.float32)

    in_specs = [
        pl.BlockSpec((d, blk),
                     lambda c, k=k: (0, jnp.minimum(c + k * tr, last)))
        for k in range(p)
    ]

    def pack(tbl_t):
        return pl.pallas_call(
            body,
            grid=(tr,),
            in_specs=in_specs,
            out_specs=pl.BlockSpec((blk, 128), lambda c: (c, 0)),
            out_shape=jax.ShapeDtypeStruct((h, 128), jnp.float32),
        )(*([tbl_t] * p))

    return pack, h


_pack_item, H_ITEM = _mk_pack(ITEM_ROWS, ITEM_D, ITEM_BLK)
_pack_brand, H_BRAND = _mk_pack(BRAND_ROWS, BRAND_D, BRAND_BLK)
_pack_cat, H_CAT = _mk_pack(CAT_ROWS, CAT_D, CAT_BLK)


def _sc_body(all_idx,
             user_table, item_pk, cat_pk, brand_pk,
             out_hbm,
             idx_v, u_rows, pi_rows, pc_rows, pb_rows,
             ni_rows, nc_rows, nb_rows, partial,
             sem):
    nc_ax = jax.lax.axis_index("c")
    ns_ax = jax.lax.axis_index("s")
    wid = ns_ax * 2 + nc_ax
    base = wid * BPW

    def chunk_body(c, _):
        pltpu.sync_copy(all_idx.at[wid, c], idx_v)
        cps = [
            pltpu.async_copy(user_table.at[idx_v.at[0]], u_rows, sem),
            pltpu.async_copy(item_pk.at[idx_v.at[1]], pi_rows, sem),
            pltpu.async_copy(cat_pk.at[idx_v.at[3]], pc_rows, sem),
            pltpu.async_copy(brand_pk.at[idx_v.at[5]], pb_rows, sem),
            pltpu.async_copy(item_pk.at[idx_v.at[7]], ni_rows, sem),
            pltpu.async_copy(cat_pk.at[idx_v.at[9]], nc_rows, sem),
            pltpu.async_copy(brand_pk.at[idx_v.at[11]], nb_rows, sem),
        ]
        for cp in cps:
            cp.wait()

        def group_body(g, _):
            b16 = g * L
            d16 = pl.ds(b16, L)
            pov = idx_v[2, d16]
            pcov = idx_v[4, d16]
            pbov = idx_v[6, d16]
            nov = idx_v[8, d16]
            ncov = idx_v[10, d16]
            nbov = idx_v[12, d16]
            for j in range(L):
                r = b16 + j
                po_j = pov[j]
                pco_j = pcov[j]
                pbo_j = pbov[j]
                no_j = nov[j]
                nco_j = ncov[j]
                nbo_j = nbov[j]
                acc = jnp.zeros((L,), jnp.float32)
                for k in range(4):
                    du = pl.ds(k * L, L)
                    pv = pi_rows[r, pl.ds(po_j + k * L, L)]
                    nv = ni_rows[r, pl.ds(no_j + k * L, L)]
                    acc += u_rows[r, du] * (nv - pv)
                for k in range(2):
                    du = pl.ds(64 + k * L, L)
                    pv = pc_rows[r, pl.ds(pco_j + k * L, L)]
                    nv = nc_rows[r, pl.ds(nco_j + k * L, L)]
                    acc += u_rows[r, du] * (nv - pv)
                for k in range(2):
                    du = pl.ds(96 + k * L, L)
                    pv = pb_rows[r, pl.ds(pbo_j + k * L, L)]
                    nv = nb_rows[r, pl.ds(nbo_j + k * L, L)]
                    acc += u_rows[r, du] * (nv - pv)
                partial[r, :] = acc
            return 0

        lax.fori_loop(0, C // L, group_body, 0)
        pltpu.sync_copy(partial, out_hbm.at[pl.ds(base + c * C, C)])
        return 0

    lax.fori_loop(0, NCH, chunk_body, 0)


@jax.jit
def _sc_scores(*args):
    mesh = plsc.VectorSubcoreMesh(core_axis_name="c", subcore_axis_name="s")
    f = functools.partial(
        pl.kernel,
        mesh=mesh,
        compiler_params=pltpu.CompilerParams(needs_layout_passes=False),
        out_type=jax.ShapeDtypeStruct((B, L), jnp.float32),
        scratch_types=[
            pltpu.VMEM((13, C), jnp.int32),
            pltpu.VMEM((C, 128), jnp.float32),
            pltpu.VMEM((C, 128), jnp.float32),
            pltpu.VMEM((C, 128), jnp.float32),
            pltpu.VMEM((C, 128), jnp.float32),
            pltpu.VMEM((C, 128), jnp.float32),
            pltpu.VMEM((C, 128), jnp.float32),
            pltpu.VMEM((C, 128), jnp.float32),
            pltpu.VMEM((C, L), jnp.float32),
            pltpu.SemaphoreType.DMA,
        ],
    )(_sc_body)
    return f(*args)


def _tc_loss_body(p_ref, o_ref):
    s = jnp.sum(p_ref[...], axis=1, keepdims=True)
    sp = jnp.maximum(s, 0.0) + jnp.log1p(jnp.exp(-jnp.abs(s)))
    o_ref[...] = (jnp.sum(sp) * (1.0 / B)).reshape(1, 1)


def _tc_loss(partials):
    out = pl.pallas_call(
        _tc_loss_body,
        out_shape=jax.ShapeDtypeStruct((1, 1), jnp.float32),
    )(partials)
    return out[0, 0]


def _split(idx, h, d):
    q = idx // h
    return idx - q * h, q * d


def kernel(user, item, item_cat, item_brand, neg_item, neg_item_cat,
           neg_item_brand, user_table, item_table, cat_table, brand_table):
    def rs(x):
        return x.astype(jnp.int32).reshape(NW, NCH, C)

    item_pk = _pack_item(item_table.T)
    cat_pk = _pack_cat(cat_table.T)
    brand_pk = _pack_brand(brand_table.T)

    pi, po = _split(item.astype(jnp.int32), H_ITEM, ITEM_D)
    pc, pco = _split(item_cat.astype(jnp.int32), H_CAT, CAT_D)
    pb, pbo = _split(item_brand.astype(jnp.int32), H_BRAND, BRAND_D)
    ni, no = _split(neg_item.astype(jnp.int32), H_ITEM, ITEM_D)
    ncc, nco = _split(neg_item_cat.astype(jnp.int32), H_CAT, CAT_D)
    nb, nbo = _split(neg_item_brand.astype(jnp.int32), H_BRAND, BRAND_D)

    all_idx = jnp.stack(
        [rs(user), rs(pi), rs(po), rs(pc), rs(pco), rs(pb), rs(pbo),
         rs(ni), rs(no), rs(ncc), rs(nco), rs(nb), rs(nbo)],
        axis=2)
    partials = _sc_scores(all_idx, user_table, item_pk, cat_pk, brand_pk)
    return _tc_loss(partials)


# speedup vs baseline: 1.5550x; 1.0150x over previous
import functools

import jax
import jax.numpy as jnp
from jax import lax
from jax.experimental import pallas as pl
from jax.experimental.pallas import tpu as pltpu
from jax.experimental.pallas import tpu_sc as plsc

B = 16384
NW = 32
BPW = B // NW
C = 64
NCH = BPW // C
L = 16

ITEM_ROWS, ITEM_D, ITEM_BLK = 1000000, 64, 16384
BRAND_ROWS, BRAND_D, BRAND_BLK = 100000, 32, 4096
CAT_ROWS, CAT_D, CAT_BLK = 1000, 32, 256


def _mk_pack(rows, d, blk):
    p = 128 // d
    tr = (rows + p * blk - 1) // (p * blk)
    h = tr * blk
    last = (rows - 1) // blk

    def body(*refs):
        x = jnp.concatenate([r[...] for r in refs[:-1]], axis=0)
        eye = jnp.eye(128, dtype=jnp.float32)
        dn = (((0,), (0,)), ((), ()))
        refs[-1][...] = lax.dot_general(x, eye, dn,
                                        preferred_element_type=jnp.float32)

    in_specs = [
        pl.BlockSpec((d, blk),
                     lambda c, k=k: (0, jnp.minimum(c + k * tr, last)))
        for k in range(p)
    ]

    def pack(tbl_t):
        return pl.pallas_call(
            body,
            grid=(tr,),
            in_specs=in_specs,
            out_specs=pl.BlockSpec((blk, 128), lambda c: (c, 0)),
            out_shape=jax.ShapeDtypeStruct((h, 128), jnp.float32),
        )(*([tbl_t] * p))

    return pack, h


_pack_item, H_ITEM = _mk_pack(ITEM_ROWS, ITEM_D, ITEM_BLK)
_pack_brand, H_BRAND = _mk_pack(BRAND_ROWS, BRAND_D, BRAND_BLK)
_pack_cat, H_CAT = _mk_pack(CAT_ROWS, CAT_D, CAT_BLK)


def _sc_body(all_idx,
             user_table, item_pk, cat_pk, brand_pk,
             out_hbm,
             idx_v, u_rows, pi_rows, pc_rows, pb_rows,
             ni_rows, nc_rows, nb_rows, partial,
             sem):
    nc_ax = jax.lax.axis_index("c")
    ns_ax = jax.lax.axis_index("s")
    wid = ns_ax * 2 + nc_ax
    base = wid * BPW

    def chunk_body(c, _):
        pltpu.sync_copy(all_idx.at[wid, c], idx_v)
        cps = [
            pltpu.async_copy(user_table.at[idx_v.at[0]], u_rows, sem),
            pltpu.async_copy(item_pk.at[idx_v.at[1]], pi_rows, sem),
            pltpu.async_copy(cat_pk.at[idx_v.at[3]], pc_rows, sem),
            pltpu.async_copy(brand_pk.at[idx_v.at[5]], pb_rows, sem),
            pltpu.async_copy(item_pk.at[idx_v.at[7]], ni_rows, sem),
            pltpu.async_copy(cat_pk.at[idx_v.at[9]], nc_rows, sem),
            pltpu.async_copy(brand_pk.at[idx_v.at[11]], nb_rows, sem),
        ]
        for cp in cps:
            cp.wait()

        def group_body(g, _):
            b16 = g * L
            d16 = pl.ds(b16, L)
            pov = idx_v[2, d16]
            pcov = idx_v[4, d16]
            pbov = idx_v[6, d16]
            nov = idx_v[8, d16]
            ncov = idx_v[10, d16]
            nbov = idx_v[12, d16]
            for j in range(L):
                r = b16 + j
                po_j = pov[j]
                pco_j = pcov[j]
                pbo_j = pbov[j]
                no_j = nov[j]
                nco_j = ncov[j]
                nbo_j = nbov[j]
                acc = jnp.zeros((L,), jnp.float32)
                for k in range(4):
                    du = pl.ds(k * L, L)
                    pv = pi_rows[r, pl.ds(po_j + k * L, L)]
                    nv = ni_rows[r, pl.ds(no_j + k * L, L)]
                    acc += u_rows[r, du] * (nv - pv)
                for k in range(2):
                    du = pl.ds(64 + k * L, L)
                    pv = pc_rows[r, pl.ds(pco_j + k * L, L)]
                    nv = nc_rows[r, pl.ds(nco_j + k * L, L)]
                    acc += u_rows[r, du] * (nv - pv)
                for k in range(2):
                    du = pl.ds(96 + k * L, L)
                    pv = pb_rows[r, pl.ds(pbo_j + k * L, L)]
                    nv = nb_rows[r, pl.ds(nbo_j + k * L, L)]
                    acc += u_rows[r, du] * (nv - pv)
                partial[r, :] = acc
            return 0

        lax.fori_loop(0, C // L, group_body, 0)
        pltpu.sync_copy(partial, out_hbm.at[pl.ds(base + c * C, C)])
        return 0

    lax.fori_loop(0, NCH, chunk_body, 0)


@jax.jit
def _sc_scores(*args):
    mesh = plsc.VectorSubcoreMesh(core_axis_name="c", subcore_axis_name="s")
    f = functools.partial(
        pl.kernel,
        mesh=mesh,
        compiler_params=pltpu.CompilerParams(needs_layout_passes=False),
        out_type=jax.ShapeDtypeStruct((B, L), jnp.float32),
        scratch_types=[
            pltpu.VMEM((13, C), jnp.int32),
            pltpu.VMEM((C, 128), jnp.float32),
            pltpu.VMEM((C, 128), jnp.float32),
            pltpu.VMEM((C, 128), jnp.float32),
            pltpu.VMEM((C, 128), jnp.float32),
            pltpu.VMEM((C, 128), jnp.float32),
            pltpu.VMEM((C, 128), jnp.float32),
            pltpu.VMEM((C, 128), jnp.float32),
            pltpu.VMEM((C, L), jnp.float32),
            pltpu.SemaphoreType.DMA,
        ],
    )(_sc_body)
    return f(*args)


def _tc_loss_body(p_ref, o_ref):
    s = jnp.sum(p_ref[...], axis=1, keepdims=True)
    sp = jnp.maximum(s, 0.0) + jnp.log1p(jnp.exp(-jnp.abs(s)))
    o_ref[...] = (jnp.sum(sp) * (1.0 / B)).reshape(1, 1)


def _tc_loss(partials):
    out = pl.pallas_call(
        _tc_loss_body,
        out_shape=jax.ShapeDtypeStruct((1, 1), jnp.float32),
    )(partials)
    return out[0, 0]


def _split(idx, h, d):
    q = idx // h
    return idx - q * h, q * d


def kernel(user, item, item_cat, item_brand, neg_item, neg_item_cat,
           neg_item_brand, user_table, item_table, cat_table, brand_table):
    def rs(x):
        return x.astype(jnp.int32).reshape(NW, NCH, C)

    item_pk = _pack_item(item_table.T)
    cat_pk = _pack_cat(cat_table.T)
    brand_pk = _pack_brand(brand_table.T)

    pi, po = _split(item.astype(jnp.int32), H_ITEM, ITEM_D)
    pc, pco = _split(item_cat.astype(jnp.int32), H_CAT, CAT_D)
    pb, pbo = _split(item_brand.astype(jnp.int32), H_BRAND, BRAND_D)
    ni, no = _split(neg_item.astype(jnp.int32), H_ITEM, ITEM_D)
    ncc, nco = _split(neg_item_cat.astype(jnp.int32), H_CAT, CAT_D)
    nb, nbo = _split(neg_item_brand.astype(jnp.int32), H_BRAND, BRAND_D)

    all_idx = jnp.stack(
        [rs(user), rs(pi), rs(po), rs(pc), rs(pco), rs(pb), rs(pbo),
         rs(ni), rs(no), rs(ncc), rs(nco), rs(nb), rs(nbo)],
        axis=2)
    partials = _sc_scores(all_idx, user_table, item_pk, cat_pk, brand_pk)
    return _tc_loss(partials)


# speedup vs baseline: 1.5595x; 1.0029x over previous
import functools

import jax
import jax.numpy as jnp
from jax import lax
from jax.experimental import pallas as pl
from jax.experimental.pallas import tpu as pltpu
from jax.experimental.pallas import tpu_sc as plsc

B = 16384
NW = 32
BPW = B // NW
C = 64
NCH = BPW // C
L = 16

ITEM_ROWS, ITEM_D, ITEM_BLK = 1000000, 64, 16384
BRAND_ROWS, BRAND_D, BRAND_BLK = 100000, 32, 8192
CAT_ROWS, CAT_D, CAT_BLK = 1000, 32, 256


def _mk_pack(rows, d, blk):
    p = 128 // d
    tr = (rows + p * blk - 1) // (p * blk)
    h = tr * blk
    last = (rows - 1) // blk

    def body(*refs):
        x = jnp.concatenate([r[...] for r in refs[:-1]], axis=0)
        eye = jnp.eye(128, dtype=jnp.float32)
        dn = (((0,), (0,)), ((), ()))
        refs[-1][...] = lax.dot_general(x, eye, dn,
                                        preferred_element_type=jnp.float32)

    in_specs = [
        pl.BlockSpec((d, blk),
                     lambda c, k=k: (0, jnp.minimum(c + k * tr, last)))
        for k in range(p)
    ]

    def pack(tbl_t):
        return pl.pallas_call(
            body,
            grid=(tr,),
            in_specs=in_specs,
            out_specs=pl.BlockSpec((blk, 128), lambda c: (c, 0)),
            out_shape=jax.ShapeDtypeStruct((h, 128), jnp.float32),
        )(*([tbl_t] * p))

    return pack, h


_pack_item, H_ITEM = _mk_pack(ITEM_ROWS, ITEM_D, ITEM_BLK)
_pack_brand, H_BRAND = _mk_pack(BRAND_ROWS, BRAND_D, BRAND_BLK)
_pack_cat, H_CAT = _mk_pack(CAT_ROWS, CAT_D, CAT_BLK)


def _sc_body(all_idx,
             user_table, item_pk, cat_pk, brand_pk,
             out_hbm,
             idx_v, u_rows, pi_rows, pc_rows, pb_rows,
             ni_rows, nc_rows, nb_rows, partial,
             sem):
    nc_ax = jax.lax.axis_index("c")
    ns_ax = jax.lax.axis_index("s")
    wid = ns_ax * 2 + nc_ax
    base = wid * BPW

    def chunk_body(c, _):
        pltpu.sync_copy(all_idx.at[wid, c], idx_v)
        cps = [
            pltpu.async_copy(user_table.at[idx_v.at[0]], u_rows, sem),
            pltpu.async_copy(item_pk.at[idx_v.at[1]], pi_rows, sem),
            pltpu.async_copy(cat_pk.at[idx_v.at[3]], pc_rows, sem),
            pltpu.async_copy(brand_pk.at[idx_v.at[5]], pb_rows, sem),
            pltpu.async_copy(item_pk.at[idx_v.at[7]], ni_rows, sem),
            pltpu.async_copy(cat_pk.at[idx_v.at[9]], nc_rows, sem),
            pltpu.async_copy(brand_pk.at[idx_v.at[11]], nb_rows, sem),
        ]
        for cp in cps:
            cp.wait()

        def group_body(g, _):
            b16 = g * L
            d16 = pl.ds(b16, L)
            pov = idx_v[2, d16]
            pcov = idx_v[4, d16]
            pbov = idx_v[6, d16]
            nov = idx_v[8, d16]
            ncov = idx_v[10, d16]
            nbov = idx_v[12, d16]
            for j in range(L):
                r = b16 + j
                po_j = pov[j]
                pco_j = pcov[j]
                pbo_j = pbov[j]
                no_j = nov[j]
                nco_j = ncov[j]
                nbo_j = nbov[j]
                acc = jnp.zeros((L,), jnp.float32)
                for k in range(4):
                    du = pl.ds(k * L, L)
                    pv = pi_rows[r, pl.ds(po_j + k * L, L)]
                    nv = ni_rows[r, pl.ds(no_j + k * L, L)]
                    acc += u_rows[r, du] * (nv - pv)
                for k in range(2):
                    du = pl.ds(64 + k * L, L)
                    pv = pc_rows[r, pl.ds(pco_j + k * L, L)]
                    nv = nc_rows[r, pl.ds(nco_j + k * L, L)]
                    acc += u_rows[r, du] * (nv - pv)
                for k in range(2):
                    du = pl.ds(96 + k * L, L)
                    pv = pb_rows[r, pl.ds(pbo_j + k * L, L)]
                    nv = nb_rows[r, pl.ds(nbo_j + k * L, L)]
                    acc += u_rows[r, du] * (nv - pv)
                partial[r, :] = acc
            return 0

        lax.fori_loop(0, C // L, group_body, 0)
        pltpu.sync_copy(partial, out_hbm.at[pl.ds(base + c * C, C)])
        return 0

    lax.fori_loop(0, NCH, chunk_body, 0)


@jax.jit
def _sc_scores(*args):
    mesh = plsc.VectorSubcoreMesh(core_axis_name="c", subcore_axis_name="s")
    f = functools.partial(
        pl.kernel,
        mesh=mesh,
        compiler_params=pltpu.CompilerParams(needs_layout_passes=False),
        out_type=jax.ShapeDtypeStruct((B, L), jnp.float32),
        scratch_types=[
            pltpu.VMEM((13, C), jnp.int32),
            pltpu.VMEM((C, 128), jnp.float32),
            pltpu.VMEM((C, 128), jnp.float32),
            pltpu.VMEM((C, 128), jnp.float32),
            pltpu.VMEM((C, 128), jnp.float32),
            pltpu.VMEM((C, 128), jnp.float32),
            pltpu.VMEM((C, 128), jnp.float32),
            pltpu.VMEM((C, 128), jnp.float32),
            pltpu.VMEM((C, L), jnp.float32),
            pltpu.SemaphoreType.DMA,
        ],
    )(_sc_body)
    return f(*args)


def _tc_loss_body(p_ref, o_ref):
    s = jnp.sum(p_ref[...], axis=1, keepdims=True)
    sp = jnp.maximum(s, 0.0) + jnp.log1p(jnp.exp(-jnp.abs(s)))
    o_ref[...] = (jnp.sum(sp) * (1.0 / B)).reshape(1, 1)


def _tc_loss(partials):
    out = pl.pallas_call(
        _tc_loss_body,
        out_shape=jax.ShapeDtypeStruct((1, 1), jnp.float32),
    )(partials)
    return out[0, 0]


def _split(idx, h, d):
    q = idx // h
    return idx - q * h, q * d


def kernel(user, item, item_cat, item_brand, neg_item, neg_item_cat,
           neg_item_brand, user_table, item_table, cat_table, brand_table):
    def rs(x):
        return x.astype(jnp.int32).reshape(NW, NCH, C)

    item_pk = _pack_item(item_table.T)
    cat_pk = _pack_cat(cat_table.T)
    brand_pk = _pack_brand(brand_table.T)

    pi, po = _split(item.astype(jnp.int32), H_ITEM, ITEM_D)
    pc, pco = _split(item_cat.astype(jnp.int32), H_CAT, CAT_D)
    pb, pbo = _split(item_brand.astype(jnp.int32), H_BRAND, BRAND_D)
    ni, no = _split(neg_item.astype(jnp.int32), H_ITEM, ITEM_D)
    ncc, nco = _split(neg_item_cat.astype(jnp.int32), H_CAT, CAT_D)
    nb, nbo = _split(neg_item_brand.astype(jnp.int32), H_BRAND, BRAND_D)

    all_idx = jnp.stack(
        [rs(user), rs(pi), rs(po), rs(pc), rs(pco), rs(pb), rs(pbo),
         rs(ni), rs(no), rs(ncc), rs(nco), rs(nb), rs(nbo)],
        axis=2)
    partials = _sc_scores(all_idx, user_table, item_pk, cat_pk, brand_pk)
    return _tc_loss(partials)
